# baseline (speedup 1.0000x reference)
import functools

import jax
import jax.numpy as jnp
from jax import lax
from jax.experimental import pallas as pl
from jax.experimental.pallas import tpu as pltpu
from jax.experimental.pallas import tpu_sc as plsc

N = 50000
E = 800000
H = 64
G = 64

NC = 2
NS = 16
L = 16

EP = 802816
EROWS = EP // 128

NPS = 51200
HH = H // NC
SUB_S = NPS // NS

W_ROWS = EROWS // (NC * NS)
S_ROWS = EROWS // NS

NPF = 50400
SUB_F = NPF // NS

_mesh = plsc.VectorSubcoreMesh(core_axis_name="c", subcore_axis_name="s")
_sc_params = pltpu.CompilerParams(
    needs_layout_passes=False, use_tc_tiling_on_sc=False)
f32 = jnp.float32
i32 = jnp.int32


def _deg_body(pk_hbm, zeros_hbm, out_hbm, slab_v, dst_v, ones_v, acc_sh):
    c = lax.axis_index("c")
    s = lax.axis_index("s")
    pltpu.sync_copy(zeros_hbm, acc_sh.at[pl.ds(s * SUB_S, SUB_S)])
    for k in range(8):
        ones_v[pl.ds(k * L, L)] = jnp.full((L,), 1.0, f32)
    w = c * NS + s
    pltpu.sync_copy(pk_hbm.at[pl.ds(w * W_ROWS * 128, W_ROWS * 128)], slab_v)
    plsc.subcore_barrier()

    def body(i, _):
        for j in range(4):
            for k in range(8):
                off = i * 512 + j * 128 + k * L
                dst_v[j, pl.ds(k * L, L)] = jnp.bitwise_and(
                    slab_v[pl.ds(off, L)], 0xFFFF)
        for j in range(4):
            pltpu.sync_copy(ones_v, acc_sh.at[dst_v.at[j]], add=True)
        return 0

    lax.fori_loop(0, W_ROWS // 4, body, 0)
    plsc.subcore_barrier()
    pltpu.sync_copy(acc_sh.at[pl.ds(s * SUB_S, SUB_S)],
                    out_hbm.at[c, pl.ds(s * SUB_S, SUB_S)])


_deg_kernel = functools.partial(
    pl.kernel, _deg_body, mesh=_mesh, compiler_params=_sc_params,
    out_type=jax.ShapeDtypeStruct((NC, NPS), f32),
    scratch_types=[
        pltpu.VMEM((W_ROWS * 128,), i32),
        pltpu.VMEM((4, 128), i32),
        pltpu.VMEM((128,), f32),
        pltpu.VMEM_SHARED((NPS,), f32),
    ],
)()


def _sagg_body(pk_hbm, tab_hbm, zeros_hbm, out_hbm,
               slab_v, dst_v, vals_v, tab_v, acc_sh):
    c = lax.axis_index("c")
    s = lax.axis_index("s")
    pltpu.sync_copy(zeros_hbm, acc_sh.at[pl.ds(s * SUB_S, SUB_S)])
    pltpu.sync_copy(tab_hbm, tab_v)
    w = c * NS + s
    pltpu.sync_copy(pk_hbm.at[pl.ds(w * W_ROWS * 128, W_ROWS * 128)], slab_v)
    plsc.subcore_barrier()

    def body(i, _):
        for j in range(4):
            for k in range(8):
                off = i * 512 + j * 128 + k * L
                pk = slab_v[pl.ds(off, L)]
                sidx = lax.shift_right_logical(pk, 16)
                vals_v[j, pl.ds(k * L, L)] = plsc.load_gather(tab_v, [sidx])
                dst_v[j, pl.ds(k * L, L)] = jnp.bitwise_and(pk, 0xFFFF)
        for j in range(4):
            pltpu.sync_copy(vals_v.at[j], acc_sh.at[dst_v.at[j]], add=True)
        return 0

    lax.fori_loop(0, W_ROWS // 4, body, 0)
    plsc.subcore_barrier()
    pltpu.sync_copy(acc_sh.at[pl.ds(s * SUB_S, SUB_S)],
                    out_hbm.at[c, pl.ds(s * SUB_S, SUB_S)])


_sagg_kernel = functools.partial(
    pl.kernel, _sagg_body, mesh=_mesh, compiler_params=_sc_params,
    out_type=jax.ShapeDtypeStruct((NC, NPS), f32),
    scratch_types=[
        pltpu.VMEM((W_ROWS * 128,), i32),
        pltpu.VMEM((4, 128), i32),
        pltpu.VMEM((4, 128), f32),
        pltpu.VMEM((NPS,), f32),
        pltpu.VMEM_SHARED((NPS,), f32),
    ],
)()


def _ragg_body(y_hbm, pk_hbm, zeros_hbm, out_hbm,
               idx_v, srca_v, dst_v, rows_v, acc_sh, sem0, sem1, isem0, isem1):
    c = lax.axis_index("c")
    s = lax.axis_index("s")
    sems = (sem0, sem1)
    isems = (isem0, isem1)
    pltpu.sync_copy(zeros_hbm, acc_sh.at[pl.ds(s * SUB_F, SUB_F)])
    cbase = c * N
    base = s * S_ROWS
    plsc.subcore_barrier()

    def idx_desc(g, b):
        return pltpu.make_async_copy(
            pk_hbm.at[pl.ds(base + g * 2, 2)],
            idx_v.at[pl.ds(b * 2, 2)], isems[b])

    def unpack(b):
        for j in range(2):
            for k in range(8):
                pk = idx_v[b * 2 + j, pl.ds(k * L, L)]
                srca_v[2 * b + j, pl.ds(k * L, L)] = lax.shift_right_logical(
                    pk, 16) + cbase
                dst_v[2 * b + j, pl.ds(k * L, L)] = jnp.bitwise_and(pk, 0xFFFF)

    def descs(b):
        return [
            pltpu.make_async_copy(
                y_hbm.at[srca_v.at[2 * b + j]],
                rows_v.at[pl.ds(b * 256 + j * 128, 128)], sems[b])
            for j in range(2)
        ]

    def drain_scatter(b):
        for de in descs(b):
            de.wait()
        for j in range(2):
            pltpu.sync_copy(rows_v.at[pl.ds(b * 256 + j * 128, 128)],
                            acc_sh.at[dst_v.at[2 * b + j]], add=True)

    for b in range(2):
        idx_desc(b, b).start()
    for b in range(2):
        idx_desc(b, b).wait()
        unpack(b)
        for de in descs(b):
            de.start()
        idx_desc(b + 2, b).start()

    def body(i, _):
        for b in range(2):
            g = i * 2 + b
            drain_scatter(b)
            idx_desc(g + 2, b).wait()
            unpack(b)
            for de in descs(b):
                de.start()
            idx_desc(g + 4, b).start()
        return 0

    nblk = S_ROWS // 2
    lax.fori_loop(0, nblk // 2 - 2, body, 0)
    for b in range(2):
        g = nblk - 4 + b
        drain_scatter(b)
        idx_desc(g + 2, b).wait()
        unpack(b)
        for de in descs(b):
            de.start()
    for b in range(2):
        drain_scatter(b)

    plsc.subcore_barrier()
    pltpu.sync_copy(acc_sh.at[pl.ds(s * SUB_F, SUB_F)],
                    out_hbm.at[c, pl.ds(s * SUB_F, SUB_F)])


_ragg_kernel = functools.partial(
    pl.kernel, _ragg_body, mesh=_mesh, compiler_params=_sc_params,
    out_type=jax.ShapeDtypeStruct((NC, NPF, HH), f32),
    scratch_types=[
        pltpu.VMEM((4, 128), i32),
        pltpu.VMEM((4, 128), i32),
        pltpu.VMEM((4, 128), i32),
        pltpu.VMEM((512, HH), f32),
        pltpu.VMEM_SHARED((NPF, HH), f32),
        pltpu.SemaphoreType.DMA,
        pltpu.SemaphoreType.DMA,
        pltpu.SemaphoreType.DMA,
        pltpu.SemaphoreType.DMA,
    ],
)()


def _t1_body(degp_ref, x_ref, dinv_ref, y1_ref):
    deg = degp_ref[0] + degp_ref[1] + 1.0
    dinv = lax.rsqrt(deg)
    dinv_ref[...] = dinv
    y1_ref[...] = dinv * x_ref[...]


def _t1(degp, xp):
    return pl.pallas_call(
        _t1_body,
        grid=(16,),
        in_specs=[
            pl.BlockSpec((2, 16, 200), lambda g: (0, g, 0)),
            pl.BlockSpec((16, 200), lambda g: (g, 0)),
        ],
        out_specs=[
            pl.BlockSpec((16, 200), lambda g: (g, 0)),
            pl.BlockSpec((16, 200), lambda g: (g, 0)),
        ],
        out_shape=[
            jax.ShapeDtypeStruct((256, 200), f32),
            jax.ShapeDtypeStruct((256, 200), f32),
        ],
    )(degp, xp)


def _t2_body(aggp_ref, y1_ref, dinv_ref, w1_ref, g1_ref, be1_ref, w2_ref,
             y2_ref, ssum_ref, ssq_ref):
    ph = pl.program_id(0)
    g = pl.program_id(1)
    sv = dinv_ref[...] * (aggp_ref[0] + aggp_ref[1] + y1_ref[...])

    @pl.when((ph == 0) & (g == 0))
    def _():
        ssum_ref[...] = jnp.zeros((1, 1), f32)
        ssq_ref[...] = jnp.zeros((1, 1), f32)

    @pl.when(ph == 0)
    def _():
        ssum_ref[...] += jnp.sum(sv).reshape(1, 1)
        ssq_ref[...] += jnp.sum(sv * sv).reshape(1, 1)

    @pl.when(ph == 1)
    def _():
        sbar = ssum_ref[0, 0] / N
        var_s = ssq_ref[0, 0] / N - sbar * sbar
        w1 = w1_ref[...]
        a = w1 * g1_ref[...] * lax.rsqrt(var_s * w1 * w1 + 1e-5)
        d = be1_ref[...] - sbar * a
        h1 = jnp.maximum(sv * a + d, 0.0)
        y2 = dinv_ref[...] * jnp.dot(
            h1, w2_ref[...], preferred_element_type=f32)
        y2_ref[0] = y2[:, :HH]
        y2_ref[1] = y2[:, HH:]


def _t2(agg_col, y1_col, dinv_col, W1, g1r, be1r, W2):
    return pl.pallas_call(
        _t2_body,
        grid=(2, 250),
        in_specs=[
            pl.BlockSpec((NC, 200, 1), lambda p, g: (0, g, 0)),
            pl.BlockSpec((200, 1), lambda p, g: (g, 0)),
            pl.BlockSpec((200, 1), lambda p, g: (g, 0)),
            pl.BlockSpec((1, H), lambda p, g: (0, 0)),
            pl.BlockSpec((1, H), lambda p, g: (0, 0)),
            pl.BlockSpec((1, H), lambda p, g: (0, 0)),
            pl.BlockSpec((H, H), lambda p, g: (0, 0)),
        ],
        out_specs=[
            pl.BlockSpec((NC, 200, HH), lambda p, g: (0, g, 0)),
            pl.BlockSpec((1, 1), lambda p, g: (0, 0)),
            pl.BlockSpec((1, 1), lambda p, g: (0, 0)),
        ],
        out_shape=[
            jax.ShapeDtypeStruct((NC, N, HH), f32),
            jax.ShapeDtypeStruct((1, 1), f32),
            jax.ShapeDtypeStruct((1, 1), f32),
        ],
    )(agg_col, y1_col, dinv_col, W1, g1r, be1r, W2)


def _zfull(aggp_ref, y_ref, dinv_ref, b_ref):
    agg = jnp.concatenate([aggp_ref[0], aggp_ref[1]], axis=1)
    y = jnp.concatenate([y_ref[0], y_ref[1]], axis=1)
    return dinv_ref[...] * (agg + y) + b_ref[...]


def _bn_coefs(ssum_ref, ssq_ref, g_ref, be_ref):
    mu = ssum_ref[...] / N
    var = ssq_ref[...] / N - mu * mu
    a = g_ref[...] * lax.rsqrt(var + 1e-5)
    return a, be_ref[...] - mu * a


def _next_body(aggp_ref, y_ref, dinv_ref, b_ref, g_ref, be_ref, w_ref,
               out_ref, ssum_ref, ssq_ref):
    ph = pl.program_id(0)
    g = pl.program_id(1)
    z = _zfull(aggp_ref, y_ref, dinv_ref, b_ref)

    @pl.when((ph == 0) & (g == 0))
    def _():
        ssum_ref[...] = jnp.zeros((1, H), f32)
        ssq_ref[...] = jnp.zeros((1, H), f32)

    @pl.when(ph == 0)
    def _():
        ssum_ref[...] += jnp.sum(z, axis=0, keepdims=True)
        ssq_ref[...] += jnp.sum(z * z, axis=0, keepdims=True)

    @pl.when(ph == 1)
    def _():
        a, d = _bn_coefs(ssum_ref, ssq_ref, g_ref, be_ref)
        h = jnp.maximum(z * a + d, 0.0)
        out = dinv_ref[...] * jnp.dot(
            h, w_ref[...], preferred_element_type=f32)
        out_ref[0] = out[:, :HH]
        out_ref[1] = out[:, HH:]


def _tnext(aggp, y, dinv_col, b_row, g_row, be_row, W):
    return pl.pallas_call(
        _next_body,
        grid=(2, 250),
        in_specs=[
            pl.BlockSpec((NC, 200, HH), lambda p, g: (0, g, 0)),
            pl.BlockSpec((NC, 200, HH), lambda p, g: (0, g, 0)),
            pl.BlockSpec((200, 1), lambda p, g: (g, 0)),
            pl.BlockSpec((1, H), lambda p, g: (0, 0)),
            pl.BlockSpec((1, H), lambda p, g: (0, 0)),
            pl.BlockSpec((1, H), lambda p, g: (0, 0)),
            pl.BlockSpec((H, H), lambda p, g: (0, 0)),
        ],
        out_specs=[
            pl.BlockSpec((NC, 200, HH), lambda p, g: (0, g, 0)),
            pl.BlockSpec((1, H), lambda p, g: (0, 0)),
            pl.BlockSpec((1, H), lambda p, g: (0, 0)),
        ],
        out_shape=[
            jax.ShapeDtypeStruct((NC, N, HH), f32),
            jax.ShapeDtypeStruct((1, H), f32),
            jax.ShapeDtypeStruct((1, H), f32),
        ],
    )(aggp, y, dinv_col, b_row, g_row, be_row, W)


def _pool_body(aggp_ref, y_ref, dinv_ref, b_ref, g_ref, be_ref, batch_ref,
               wl_ref, bl_ref, out_ref, ssum_ref, ssq_ref,
               pooled_ref, cnt_ref):
    ph = pl.program_id(0)
    g = pl.program_id(1)
    z = _zfull(aggp_ref, y_ref, dinv_ref, b_ref)

    @pl.when((ph == 0) & (g == 0))
    def _():
        ssum_ref[...] = jnp.zeros((1, H), f32)
        ssq_ref[...] = jnp.zeros((1, H), f32)

    @pl.when(ph == 0)
    def _():
        ssum_ref[...] += jnp.sum(z, axis=0, keepdims=True)
        ssq_ref[...] += jnp.sum(z * z, axis=0, keepdims=True)

    @pl.when(ph == 1)
    def _():
        a, d = _bn_coefs(ssum_ref, ssq_ref, g_ref, be_ref)
        h = jnp.maximum(z * a + d, 0.0)
        gids = lax.broadcasted_iota(i32, (1, G), 1)
        oh = (batch_ref[...] == gids).astype(f32)

        @pl.when(g == 0)
        def _():
            pooled_ref[...] = jnp.zeros((G, H), f32)
            cnt_ref[...] = jnp.zeros((G, 1), f32)

        pooled_ref[...] += lax.dot_general(
            oh, h, (((0,), (0,)), ((), ())), preferred_element_type=f32)
        cnt_ref[...] += lax.dot_general(
            oh, jnp.ones((200, 1), f32), (((0,), (0,)), ((), ())),
            preferred_element_type=f32)

        @pl.when(g == 249)
        def _():
            mean = pooled_ref[...] / jnp.maximum(cnt_ref[...], 1.0)
            out_ref[...] = jnp.dot(
                mean, wl_ref[...], preferred_element_type=f32) + bl_ref[...]


def _tpool(aggp, y, dinv_col, b_row, g_row, be_row, batch_col, Wl, bl_row):
    return pl.pallas_call(
        _pool_body,
        grid=(2, 250),
        in_specs=[
            pl.BlockSpec((NC, 200, HH), lambda p, g: (0, g, 0)),
            pl.BlockSpec((NC, 200, HH), lambda p, g: (0, g, 0)),
            pl.BlockSpec((200, 1), lambda p, g: (g, 0)),
            pl.BlockSpec((1, H), lambda p, g: (0, 0)),
            pl.BlockSpec((1, H), lambda p, g: (0, 0)),
            pl.BlockSpec((1, H), lambda p, g: (0, 0)),
            pl.BlockSpec((200, 1), lambda p, g: (g, 0)),
            pl.BlockSpec((H, 2), lambda p, g: (0, 0)),
            pl.BlockSpec((1, 2), lambda p, g: (0, 0)),
        ],
        out_specs=pl.BlockSpec((G, 2), lambda p, g: (0, 0)),
        out_shape=jax.ShapeDtypeStruct((G, 2), f32),
        scratch_shapes=[
            pltpu.VMEM((1, H), f32),
            pltpu.VMEM((1, H), f32),
            pltpu.VMEM((G, H), f32),
            pltpu.VMEM((G, 1), f32),
        ],
    )(aggp, y, dinv_col, b_row, g_row, be_row, batch_col, Wl, bl_row)


def kernel(x, edge_index, batch, W1, b1, g1, be1, W2, b2, g2, be2,
           W3, b3, g3, be3, Wl, bl):
    pad = EP - E
    srcp = jnp.concatenate([edge_index[0], jnp.zeros((pad,), i32)])
    dstp = jnp.concatenate([edge_index[1], jnp.full((pad,), N, i32)])
    packed = jnp.bitwise_or(lax.shift_left(srcp, 16), dstp)
    packed2d = packed.reshape(EROWS, 128)
    zeros_s = jnp.zeros((SUB_S,), f32)
    zeros_f = jnp.zeros((SUB_F, HH), f32)
    xp = jnp.concatenate([x[:, 0], jnp.zeros((NPS - N,), f32)]).reshape(256, 200)

    degp = _deg_kernel(packed, zeros_s).reshape(NC, 256, 200)
    dinv, y1 = _t1(degp, xp)

    aggp1 = _sagg_kernel(packed, y1.reshape(NPS), zeros_s)
    agg_col = aggp1.reshape(NC, NPS, 1)[:, :N]
    y1_col = y1.reshape(NPS, 1)[:N]
    dinv_col = dinv.reshape(NPS, 1)[:N]
    g1r = g1.reshape(1, H)
    be1r = be1.reshape(1, H)
    y2, _, _ = _t2(agg_col, y1_col, dinv_col, W1, g1r, be1r, W2)

    aggp2 = _ragg_kernel(y2.reshape(NC * N, HH), packed2d, zeros_f)
    y3, _, _ = _tnext(aggp2, y2, dinv_col, b2.reshape(1, H),
                      g2.reshape(1, H), be2.reshape(1, H), W3)

    aggp3 = _ragg_kernel(y3.reshape(NC * N, HH), packed2d, zeros_f)
    return _tpool(aggp3, y3, dinv_col, b3.reshape(1, H), g3.reshape(1, H),
                  be3.reshape(1, H), batch.reshape(N, 1), Wl,
                  bl.reshape(1, 2))


# speedup vs baseline: 1.7938x; 1.7938x over previous
import functools

import jax
import jax.numpy as jnp
from jax import lax
from jax.experimental import pallas as pl
from jax.experimental.pallas import tpu as pltpu
from jax.experimental.pallas import tpu_sc as plsc

N = 50000
E = 800000
H = 64
G = 64

NC = 2
NS = 16
L = 16

EP = 802816
EROWS = EP // 128

NPS = 51200
HH = H // NC
SUB_S = NPS // NS

W_ROWS = EROWS // (NC * NS)
S_ROWS = EROWS // NS

NPF = 50400
SUB_F = NPF // NS

_mesh = plsc.VectorSubcoreMesh(core_axis_name="c", subcore_axis_name="s")
_sc_params = pltpu.CompilerParams(
    needs_layout_passes=False, use_tc_tiling_on_sc=False)
f32 = jnp.float32
i32 = jnp.int32


def _writeback_s(acc_sh, out_hbm, c, s):
    @pl.when(s < NS - 1)
    def _():
        pltpu.sync_copy(acc_sh.at[pl.ds(s * SUB_S, SUB_S)],
                        out_hbm.at[c, pl.ds(s * SUB_S, SUB_S)])

    @pl.when(s == NS - 1)
    def _():
        pltpu.sync_copy(acc_sh.at[pl.ds((NS - 1) * SUB_S, N - (NS - 1) * SUB_S)],
                        out_hbm.at[c, pl.ds((NS - 1) * SUB_S, N - (NS - 1) * SUB_S)])


def _writeback_f(acc_sh, out_hbm, c, s):
    @pl.when(s < NS - 1)
    def _():
        pltpu.sync_copy(acc_sh.at[pl.ds(s * SUB_F, SUB_F)],
                        out_hbm.at[c, pl.ds(s * SUB_F, SUB_F)])

    @pl.when(s == NS - 1)
    def _():
        pltpu.sync_copy(acc_sh.at[pl.ds((NS - 1) * SUB_F, N - (NS - 1) * SUB_F)],
                        out_hbm.at[c, pl.ds((NS - 1) * SUB_F, N - (NS - 1) * SUB_F)])


def _deg_body(pk_hbm, zeros_hbm, out_hbm, slab_v, dst_v, ones_v, acc_sh):
    c = lax.axis_index("c")
    s = lax.axis_index("s")
    pltpu.sync_copy(zeros_hbm, acc_sh.at[pl.ds(s * SUB_S, SUB_S)])
    for k in range(8):
        ones_v[pl.ds(k * L, L)] = jnp.full((L,), 1.0, f32)
    w = c * NS + s
    pltpu.sync_copy(pk_hbm.at[pl.ds(w * W_ROWS * 128, W_ROWS * 128)], slab_v)
    plsc.subcore_barrier()

    def body(i, _):
        for j in range(4):
            for k in range(8):
                off = i * 512 + j * 128 + k * L
                dst_v[j, pl.ds(k * L, L)] = jnp.bitwise_and(
                    slab_v[pl.ds(off, L)], 0xFFFF)
        for j in range(4):
            pltpu.sync_copy(ones_v, acc_sh.at[dst_v.at[j]], add=True)
        return 0

    lax.fori_loop(0, W_ROWS // 4, body, 0)
    plsc.subcore_barrier()
    _writeback_s(acc_sh, out_hbm, c, s)


_deg_kernel = functools.partial(
    pl.kernel, _deg_body, mesh=_mesh, compiler_params=_sc_params,
    out_type=jax.ShapeDtypeStruct((NC, N), f32),
    scratch_types=[
        pltpu.VMEM((W_ROWS * 128,), i32),
        pltpu.VMEM((4, 128), i32),
        pltpu.VMEM((128,), f32),
        pltpu.VMEM_SHARED((NPS,), f32),
    ],
)()


def _sagg_body(pk_hbm, tab_hbm, zeros_hbm, out_hbm,
               slab_v, dst_v, vals_v, tab_v, acc_sh):
    c = lax.axis_index("c")
    s = lax.axis_index("s")
    pltpu.sync_copy(zeros_hbm, acc_sh.at[pl.ds(s * SUB_S, SUB_S)])
    pltpu.sync_copy(tab_hbm, tab_v)
    w = c * NS + s
    pltpu.sync_copy(pk_hbm.at[pl.ds(w * W_ROWS * 128, W_ROWS * 128)], slab_v)
    plsc.subcore_barrier()

    def body(i, _):
        for j in range(4):
            for k in range(8):
                off = i * 512 + j * 128 + k * L
                pk = slab_v[pl.ds(off, L)]
                sidx = lax.shift_right_logical(pk, 16)
                vals_v[j, pl.ds(k * L, L)] = plsc.load_gather(tab_v, [sidx])
                dst_v[j, pl.ds(k * L, L)] = jnp.bitwise_and(pk, 0xFFFF)
        for j in range(4):
            pltpu.sync_copy(vals_v.at[j], acc_sh.at[dst_v.at[j]], add=True)
        return 0

    lax.fori_loop(0, W_ROWS // 4, body, 0)
    plsc.subcore_barrier()
    _writeback_s(acc_sh, out_hbm, c, s)


_sagg_kernel = functools.partial(
    pl.kernel, _sagg_body, mesh=_mesh, compiler_params=_sc_params,
    out_type=jax.ShapeDtypeStruct((NC, N), f32),
    scratch_types=[
        pltpu.VMEM((W_ROWS * 128,), i32),
        pltpu.VMEM((4, 128), i32),
        pltpu.VMEM((4, 128), f32),
        pltpu.VMEM((N,), f32),
        pltpu.VMEM_SHARED((NPS,), f32),
    ],
)()


def _ragg_body(y_hbm, pk_hbm, zeros_hbm, out_hbm,
               idx_v, srca_v, dst_v, rows_v, acc_sh, sem0, sem1, isem0, isem1):
    c = lax.axis_index("c")
    s = lax.axis_index("s")
    sems = (sem0, sem1)
    isems = (isem0, isem1)
    pltpu.sync_copy(zeros_hbm, acc_sh.at[pl.ds(s * SUB_F, SUB_F)])
    cbase = c * N
    base = s * S_ROWS
    plsc.subcore_barrier()

    def idx_desc(g, b):
        return pltpu.make_async_copy(
            pk_hbm.at[pl.ds(base + g * 2, 2)],
            idx_v.at[pl.ds(b * 2, 2)], isems[b])

    def unpack(b):
        for j in range(2):
            for k in range(8):
                pk = idx_v[b * 2 + j, pl.ds(k * L, L)]
                srca_v[2 * b + j, pl.ds(k * L, L)] = lax.shift_right_logical(
                    pk, 16) + cbase
                dst_v[2 * b + j, pl.ds(k * L, L)] = jnp.bitwise_and(pk, 0xFFFF)

    def descs(b):
        return [
            pltpu.make_async_copy(
                y_hbm.at[srca_v.at[2 * b + j]],
                rows_v.at[pl.ds(b * 256 + j * 128, 128)], sems[b])
            for j in range(2)
        ]

    def drain_scatter(b):
        for de in descs(b):
            de.wait()
        for j in range(2):
            pltpu.sync_copy(rows_v.at[pl.ds(b * 256 + j * 128, 128)],
                            acc_sh.at[dst_v.at[2 * b + j]], add=True)

    for b in range(2):
        idx_desc(b, b).start()
    for b in range(2):
        idx_desc(b, b).wait()
        unpack(b)
        for de in descs(b):
            de.start()
        idx_desc(b + 2, b).start()

    def body(i, _):
        for b in range(2):
            g = i * 2 + b
            drain_scatter(b)
            idx_desc(g + 2, b).wait()
            unpack(b)
            for de in descs(b):
                de.start()
            idx_desc(g + 4, b).start()
        return 0

    nblk = S_ROWS // 2
    lax.fori_loop(0, nblk // 2 - 2, body, 0)
    for b in range(2):
        g = nblk - 4 + b
        drain_scatter(b)
        idx_desc(g + 2, b).wait()
        unpack(b)
        for de in descs(b):
            de.start()
    for b in range(2):
        drain_scatter(b)

    plsc.subcore_barrier()
    _writeback_f(acc_sh, out_hbm, c, s)


_ragg_kernel = functools.partial(
    pl.kernel, _ragg_body, mesh=_mesh, compiler_params=_sc_params,
    out_type=jax.ShapeDtypeStruct((NC, N, HH), f32),
    scratch_types=[
        pltpu.VMEM((4, 128), i32),
        pltpu.VMEM((4, 128), i32),
        pltpu.VMEM((4, 128), i32),
        pltpu.VMEM((512, HH), f32),
        pltpu.VMEM_SHARED((NPF, HH), f32),
        pltpu.SemaphoreType.DMA,
        pltpu.SemaphoreType.DMA,
        pltpu.SemaphoreType.DMA,
        pltpu.SemaphoreType.DMA,
    ],
)()


def _t1_body(degp_ref, x_ref, dinv_ref, y1_ref):
    deg = degp_ref[0] + degp_ref[1] + 1.0
    dinv = lax.rsqrt(deg)
    dinv_ref[...] = dinv
    y1_ref[...] = dinv * x_ref[...]


def _t1(degp, xp):
    return pl.pallas_call(
        _t1_body,
        out_shape=[
            jax.ShapeDtypeStruct((250, 200), f32),
            jax.ShapeDtypeStruct((250, 200), f32),
        ],
    )(degp, xp)


def _t2_body(aggp_ref, y1_ref, dinv_ref, w1_ref, g1_ref, be1_ref, w2_ref,
             y2_ref, ssum_ref, ssq_ref):
    ph = pl.program_id(0)
    g = pl.program_id(1)
    sv = dinv_ref[...] * (aggp_ref[0] + aggp_ref[1] + y1_ref[...])

    @pl.when((ph == 0) & (g == 0))
    def _():
        ssum_ref[...] = jnp.zeros((1, 1), f32)
        ssq_ref[...] = jnp.zeros((1, 1), f32)

    @pl.when(ph == 0)
    def _():
        ssum_ref[...] += jnp.sum(sv).reshape(1, 1)
        ssq_ref[...] += jnp.sum(sv * sv).reshape(1, 1)

    @pl.when(ph == 1)
    def _():
        sbar = ssum_ref[0, 0] / N
        var_s = ssq_ref[0, 0] / N - sbar * sbar
        w1 = w1_ref[...]
        a = w1 * g1_ref[...] * lax.rsqrt(var_s * w1 * w1 + 1e-5)
        d = be1_ref[...] - sbar * a
        h1 = jnp.maximum(sv * a + d, 0.0)
        y2 = dinv_ref[...] * jnp.dot(
            h1, w2_ref[...], preferred_element_type=f32)
        y2_ref[0] = y2[:, :HH]
        y2_ref[1] = y2[:, HH:]


def _t2(agg_col, y1_col, dinv_col, W1, g1r, be1r, W2):
    return pl.pallas_call(
        _t2_body,
        grid=(2, 10),
        in_specs=[
            pl.BlockSpec((NC, 5000, 1), lambda p, g: (0, g, 0)),
            pl.BlockSpec((5000, 1), lambda p, g: (g, 0)),
            pl.BlockSpec((5000, 1), lambda p, g: (g, 0)),
            pl.BlockSpec((1, H), lambda p, g: (0, 0)),
            pl.BlockSpec((1, H), lambda p, g: (0, 0)),
            pl.BlockSpec((1, H), lambda p, g: (0, 0)),
            pl.BlockSpec((H, H), lambda p, g: (0, 0)),
        ],
        out_specs=[
            pl.BlockSpec((NC, 5000, HH), lambda p, g: (0, g, 0)),
            pl.BlockSpec((1, 1), lambda p, g: (0, 0)),
            pl.BlockSpec((1, 1), lambda p, g: (0, 0)),
        ],
        out_shape=[
            jax.ShapeDtypeStruct((NC, N, HH), f32),
            jax.ShapeDtypeStruct((1, 1), f32),
            jax.ShapeDtypeStruct((1, 1), f32),
        ],
    )(agg_col, y1_col, dinv_col, W1, g1r, be1r, W2)


def _zfull(aggp_ref, y_ref, dinv_ref, b_ref):
    agg = jnp.concatenate([aggp_ref[0], aggp_ref[1]], axis=1)
    y = jnp.concatenate([y_ref[0], y_ref[1]], axis=1)
    return dinv_ref[...] * (agg + y) + b_ref[...]


def _bn_coefs(ssum_ref, ssq_ref, g_ref, be_ref):
    mu = ssum_ref[...] / N
    var = ssq_ref[...] / N - mu * mu
    a = g_ref[...] * lax.rsqrt(var + 1e-5)
    return a, be_ref[...] - mu * a


def _next_body(aggp_ref, y_ref, dinv_ref, b_ref, g_ref, be_ref, w_ref,
               out_ref, ssum_ref, ssq_ref):
    ph = pl.program_id(0)
    g = pl.program_id(1)
    z = _zfull(aggp_ref, y_ref, dinv_ref, b_ref)

    @pl.when((ph == 0) & (g == 0))
    def _():
        ssum_ref[...] = jnp.zeros((1, H), f32)
        ssq_ref[...] = jnp.zeros((1, H), f32)

    @pl.when(ph == 0)
    def _():
        ssum_ref[...] += jnp.sum(z, axis=0, keepdims=True)
        ssq_ref[...] += jnp.sum(z * z, axis=0, keepdims=True)

    @pl.when(ph == 1)
    def _():
        a, d = _bn_coefs(ssum_ref, ssq_ref, g_ref, be_ref)
        h = jnp.maximum(z * a + d, 0.0)
        out = dinv_ref[...] * jnp.dot(
            h, w_ref[...], preferred_element_type=f32)
        out_ref[0] = out[:, :HH]
        out_ref[1] = out[:, HH:]


def _tnext(aggp, y, dinv_col, b_row, g_row, be_row, W):
    return pl.pallas_call(
        _next_body,
        grid=(2, 10),
        in_specs=[
            pl.BlockSpec((NC, 5000, HH), lambda p, g: (0, g, 0)),
            pl.BlockSpec((NC, 5000, HH), lambda p, g: (0, g, 0)),
            pl.BlockSpec((5000, 1), lambda p, g: (g, 0)),
            pl.BlockSpec((1, H), lambda p, g: (0, 0)),
            pl.BlockSpec((1, H), lambda p, g: (0, 0)),
            pl.BlockSpec((1, H), lambda p, g: (0, 0)),
            pl.BlockSpec((H, H), lambda p, g: (0, 0)),
        ],
        out_specs=[
            pl.BlockSpec((NC, 5000, HH), lambda p, g: (0, g, 0)),
            pl.BlockSpec((1, H), lambda p, g: (0, 0)),
            pl.BlockSpec((1, H), lambda p, g: (0, 0)),
        ],
        out_shape=[
            jax.ShapeDtypeStruct((NC, N, HH), f32),
            jax.ShapeDtypeStruct((1, H), f32),
            jax.ShapeDtypeStruct((1, H), f32),
        ],
    )(aggp, y, dinv_col, b_row, g_row, be_row, W)


def _pool_body(aggp_ref, y_ref, dinv_ref, b_ref, g_ref, be_ref, batch_ref,
               wl_ref, bl_ref, out_ref, ssum_ref, ssq_ref,
               pooled_ref, cnt_ref):
    ph = pl.program_id(0)
    g = pl.program_id(1)
    z = _zfull(aggp_ref, y_ref, dinv_ref, b_ref)

    @pl.when((ph == 0) & (g == 0))
    def _():
        ssum_ref[...] = jnp.zeros((1, H), f32)
        ssq_ref[...] = jnp.zeros((1, H), f32)

    @pl.when(ph == 0)
    def _():
        ssum_ref[...] += jnp.sum(z, axis=0, keepdims=True)
        ssq_ref[...] += jnp.sum(z * z, axis=0, keepdims=True)

    @pl.when(ph == 1)
    def _():
        a, d = _bn_coefs(ssum_ref, ssq_ref, g_ref, be_ref)
        h = jnp.maximum(z * a + d, 0.0)
        gids = lax.broadcasted_iota(i32, (1, G), 1)
        oh = (batch_ref[...] == gids).astype(f32)

        @pl.when(g == 0)
        def _():
            pooled_ref[...] = jnp.zeros((G, H), f32)
            cnt_ref[...] = jnp.zeros((G, 1), f32)

        pooled_ref[...] += lax.dot_general(
            oh, h, (((0,), (0,)), ((), ())), preferred_element_type=f32)
        cnt_ref[...] += lax.dot_general(
            oh, jnp.ones((5000, 1), f32), (((0,), (0,)), ((), ())),
            preferred_element_type=f32)

        @pl.when(g == 9)
        def _():
            mean = pooled_ref[...] / jnp.maximum(cnt_ref[...], 1.0)
            out_ref[...] = jnp.dot(
                mean, wl_ref[...], preferred_element_type=f32) + bl_ref[...]


def _tpool(aggp, y, dinv_col, b_row, g_row, be_row, batch_col, Wl, bl_row):
    return pl.pallas_call(
        _pool_body,
        grid=(2, 10),
        in_specs=[
            pl.BlockSpec((NC, 5000, HH), lambda p, g: (0, g, 0)),
            pl.BlockSpec((NC, 5000, HH), lambda p, g: (0, g, 0)),
            pl.BlockSpec((5000, 1), lambda p, g: (g, 0)),
            pl.BlockSpec((1, H), lambda p, g: (0, 0)),
            pl.BlockSpec((1, H), lambda p, g: (0, 0)),
            pl.BlockSpec((1, H), lambda p, g: (0, 0)),
            pl.BlockSpec((5000, 1), lambda p, g: (g, 0)),
            pl.BlockSpec((H, 2), lambda p, g: (0, 0)),
            pl.BlockSpec((1, 2), lambda p, g: (0, 0)),
        ],
        out_specs=pl.BlockSpec((G, 2), lambda p, g: (0, 0)),
        out_shape=jax.ShapeDtypeStruct((G, 2), f32),
        scratch_shapes=[
            pltpu.VMEM((1, H), f32),
            pltpu.VMEM((1, H), f32),
            pltpu.VMEM((G, H), f32),
            pltpu.VMEM((G, 1), f32),
        ],
    )(aggp, y, dinv_col, b_row, g_row, be_row, batch_col, Wl, bl_row)


def kernel(x, edge_index, batch, W1, b1, g1, be1, W2, b2, g2, be2,
           W3, b3, g3, be3, Wl, bl):
    pad = EP - E
    srcp = jnp.concatenate([edge_index[0], jnp.zeros((pad,), i32)])
    dstp = jnp.concatenate([edge_index[1], jnp.full((pad,), N, i32)])
    packed = jnp.bitwise_or(lax.shift_left(srcp, 16), dstp)
    packed2d = packed.reshape(EROWS, 128)
    zeros_s = jnp.zeros((SUB_S,), f32)
    zeros_f = jnp.zeros((SUB_F, HH), f32)
    xp = x.reshape(250, 200)

    degp = _deg_kernel(packed, zeros_s).reshape(NC, 250, 200)
    dinv, y1 = _t1(degp, xp)

    agg = _sagg_kernel(packed, y1.reshape(N), zeros_s)
    agg_col = agg.reshape(NC, N, 1)
    y1_col = y1.reshape(N, 1)
    dinv_col = dinv.reshape(N, 1)
    y2, _, _ = _t2(agg_col, y1_col, dinv_col, W1, g1.reshape(1, H),
                   be1.reshape(1, H), W2)

    aggp2 = _ragg_kernel(y2.reshape(NC * N, HH), packed2d, zeros_f)
    y3, _, _ = _tnext(aggp2, y2, dinv_col, b2.reshape(1, H),
                      g2.reshape(1, H), be2.reshape(1, H), W3)

    aggp3 = _ragg_kernel(y3.reshape(NC * N, HH), packed2d, zeros_f)
    return _tpool(aggp3, y3, dinv_col, b3.reshape(1, H), g3.reshape(1, H),
                  be3.reshape(1, H), batch.reshape(N, 1), Wl,
                  bl.reshape(1, 2))
